# baseline (speedup 1.0000x reference)
import functools

import jax
import jax.numpy as jnp
import numpy as np
from jax import lax
from jax.experimental import pallas as pl
from jax.experimental.pallas import tpu as pltpu
from jax.experimental.pallas import tpu_sc as plsc

N = 10000
E = 320000
H = 128
TP = 3
C = 128
NCH = 2560
EP = NCH * C
CPT = NCH // 16
QCP = CPT // 4
NP = 10112
ZROWS = NP // 16
DEGL = 32768
DEGT = DEGL // 16

BLK = 1000
NB = N // BLK

BETA1 = float(np.log(0.5 / 1.0 + 1.0))
BETA2 = float(np.log(0.5 / 2.0 + 1.0))

_MESH = plsc.VectorSubcoreMesh(core_axis_name="c", subcore_axis_name="s")


def _sc_degrees(dst_deg, ones_c, zeros_deg):
    @functools.partial(
        pl.kernel,
        out_type=jax.ShapeDtypeStruct((2, 1, DEGL), jnp.float32),
        mesh=_MESH,
        scratch_types=[
            pltpu.VMEM((NCH // 32, C), jnp.int32),
            pltpu.VMEM((C,), jnp.float32),
            pltpu.VMEM_SHARED((DEGL,), jnp.float32),
            pltpu.SemaphoreType.DMA,
        ],
    )
    def k(dst_hbm, ones_hbm, zero_hbm, out_hbm, idx_v, ones_v, acc, sem):
        cid = lax.axis_index("c")
        sid = lax.axis_index("s")
        wid = sid * 2 + cid
        cpt = NCH // 32
        pltpu.sync_copy(ones_hbm, ones_v)
        pltpu.sync_copy(zero_hbm, acc.at[pl.ds(sid * DEGT, DEGT)])
        plsc.subcore_barrier()
        for t in range(TP):
            pltpu.sync_copy(dst_hbm.at[t, pl.ds(wid * cpt, cpt)], idx_v)

            def body(g, carry):
                pltpu.sync_copy(ones_v, acc.at[idx_v.at[g]], add=True)
                return carry

            lax.fori_loop(0, cpt, body, 0)
        plsc.subcore_barrier()
        pltpu.sync_copy(acc.at[pl.ds(sid * DEGT, DEGT)],
                        out_hbm.at[cid, 0, pl.ds(sid * DEGT, DEGT)])

    return k(dst_deg, ones_c, zeros_deg)


def _sc_segsum(vals_flat, src_g, dst_g, zeros_rows):
    @functools.partial(
        pl.kernel,
        out_type=jax.ShapeDtypeStruct((1, TP, NP, H), jnp.float32),
        mesh=_MESH,
        scratch_types=[
            pltpu.VMEM((QCP, C), jnp.int32),
            pltpu.VMEM((QCP, C), jnp.int32),
            pltpu.VMEM((C, H), jnp.float32),
            pltpu.VMEM((C, H), jnp.float32),
            pltpu.VMEM_SHARED((NP, H), jnp.float32),
            pltpu.SemaphoreType.DMA,
            pltpu.SemaphoreType.DMA,
        ],
    )
    def k(vals_hbm, src_hbm, dst_hbm, zero_hbm, out_hbm,
          src_v, dst_v, rows0, rows1, acc, sem0, sem1):
        cid = lax.axis_index("c")
        sid = lax.axis_index("s")

        def run_quarter(t, base):
            pltpu.sync_copy(src_hbm.at[t, pl.ds(base, QCP)], src_v)
            pltpu.sync_copy(dst_hbm.at[t, pl.ds(base, QCP)], dst_v)
            pltpu.async_copy(vals_hbm.at[src_v.at[0]], rows0, sem0)

            def body(gp, carry):
                g0 = 2 * gp
                pltpu.async_copy(vals_hbm.at[src_v.at[g0 + 1]], rows1, sem1)
                pltpu.make_async_copy(vals_hbm.at[src_v.at[g0]], rows0,
                                      sem0).wait()
                pltpu.sync_copy(rows0, acc.at[dst_v.at[g0]], add=True)

                @pl.when(gp + 1 < QCP // 2)
                def _():
                    pltpu.async_copy(vals_hbm.at[src_v.at[g0 + 2]], rows0,
                                     sem0)

                pltpu.make_async_copy(vals_hbm.at[src_v.at[g0 + 1]], rows1,
                                      sem1).wait()
                pltpu.sync_copy(rows1, acc.at[dst_v.at[g0 + 1]], add=True)
                return carry

            lax.fori_loop(0, QCP // 2, body, 0)

        @pl.when(cid == 0)
        def _():
            for t in range(TP):
                pltpu.sync_copy(zero_hbm, acc.at[pl.ds(sid * ZROWS, ZROWS)])
                plsc.subcore_barrier()
                for q in range(4):
                    run_quarter(t, sid * CPT + q * QCP)
                plsc.subcore_barrier()
                pltpu.sync_copy(acc.at[pl.ds(sid * ZROWS, ZROWS)],
                                out_hbm.at[0, t, pl.ds(sid * ZROWS, ZROWS)])
                plsc.subcore_barrier()

    return k(vals_flat, src_g, dst_g, zeros_rows)


def _tc_weights(w1_0, w1_1, r1_WihT, r2_WihT, b1sum, b2sum):
    def body(w0_ref, w1_ref, wt1_ref, wt2_ref, b1_ref, b2_ref, o_ref):
        for li, (w0, wt, bs) in enumerate(
                ((w0_ref[...], wt1_ref, b1_ref),
                 (w1_ref[...], wt2_ref, b2_ref))):
            w = w0
            c = jnp.zeros((H, H), jnp.float32)
            for t in range(TP):
                g = jnp.dot(w, wt[...], preferred_element_type=jnp.float32) + bs[...]
                i = jax.nn.sigmoid(g[:, 0:H])
                f = jax.nn.sigmoid(g[:, H:2 * H])
                gg = jnp.tanh(g[:, 2 * H:3 * H])
                o = jax.nn.sigmoid(g[:, 3 * H:4 * H])
                c = f * c + i * gg
                w = o * jnp.tanh(c)
                o_ref[li, t] = w

    return pl.pallas_call(
        body,
        out_shape=jax.ShapeDtypeStruct((2, TP, H, H), jnp.float32),
    )(w1_0, w1_1, r1_WihT, r2_WihT, b1sum, b2sum)


def _tc_stage_a(x3, Wg, dinv3):
    def body(x_ref, w_ref, d_ref, o_ref):
        o_ref[0] = jnp.dot(x_ref[0], w_ref[...],
                           preferred_element_type=jnp.float32) * d_ref[0]

    return pl.pallas_call(
        body,
        grid=(TP, NB),
        in_specs=[
            pl.BlockSpec((1, BLK, H), lambda t, b: (t, b, 0)),
            pl.BlockSpec((H, H), lambda t, b: (0, 0)),
            pl.BlockSpec((1, BLK, 1), lambda t, b: (t, b, 0)),
        ],
        out_specs=pl.BlockSpec((1, BLK, H), lambda t, b: (t, b, 0)),
        out_shape=jax.ShapeDtypeStruct((TP, N, H), jnp.float32),
    )(x3, Wg, dinv3)


def _tc_stage_b(Sp, hs, dinv3, bias):
    def body(s_ref, h_ref, d_ref, b_ref, o_ref):
        o_ref[0] = (s_ref[0, 0] + h_ref[0]) * d_ref[0] + b_ref[...]

    return pl.pallas_call(
        body,
        grid=(TP, NB),
        in_specs=[
            pl.BlockSpec((1, 1, BLK, H), lambda t, b: (0, t, b, 0)),
            pl.BlockSpec((1, BLK, H), lambda t, b: (t, b, 0)),
            pl.BlockSpec((1, BLK, 1), lambda t, b: (t, b, 0)),
            pl.BlockSpec((1, H), lambda t, b: (0, 0)),
        ],
        out_specs=pl.BlockSpec((1, BLK, H), lambda t, b: (t, b, 0)),
        out_shape=jax.ShapeDtypeStruct((TP, N, H), jnp.float32),
    )(Sp, hs, dinv3, bias)


def _tc_stage_c1(Ap, z, w1t):
    def body(a_ref, z_ref, w_ref, o_ref, ps_ref, pss_ref):
        zv = z_ref[0]
        out1 = 0.9 * a_ref[0, 0] + 0.1 * zv
        z1 = (1.0 - BETA1) * out1 + BETA1 * jnp.dot(
            out1, w_ref[0], preferred_element_type=jnp.float32)
        o_ref[0] = z1
        ps_ref[0, 0, 0] = jnp.sum(z1, axis=0)
        pss_ref[0, 0, 0] = jnp.sum(z1 * z1, axis=0)

    return pl.pallas_call(
        body,
        grid=(TP, NB),
        in_specs=[
            pl.BlockSpec((1, 1, BLK, H), lambda t, b: (0, t, b, 0)),
            pl.BlockSpec((1, BLK, H), lambda t, b: (t, b, 0)),
            pl.BlockSpec((1, H, H), lambda t, b: (t, 0, 0)),
        ],
        out_specs=[
            pl.BlockSpec((1, BLK, H), lambda t, b: (t, b, 0)),
            pl.BlockSpec((1, 1, 1, H), lambda t, b: (t, b, 0, 0)),
            pl.BlockSpec((1, 1, 1, H), lambda t, b: (t, b, 0, 0)),
        ],
        out_shape=[
            jax.ShapeDtypeStruct((TP, N, H), jnp.float32),
            jax.ShapeDtypeStruct((TP, NB, 1, H), jnp.float32),
            jax.ShapeDtypeStruct((TP, NB, 1, H), jnp.float32),
        ],
    )(Ap, z, w1t)


def _tc_stage_c2(z1, mu, var, gamma, beta):
    def body(z_ref, m_ref, v_ref, g_ref, b_ref, o_ref):
        scale = lax.rsqrt(v_ref[0] + 1e-5) * g_ref[...]
        o_ref[0] = jax.nn.relu((z_ref[0] - m_ref[0]) * scale + b_ref[...])

    return pl.pallas_call(
        body,
        grid=(TP, NB),
        in_specs=[
            pl.BlockSpec((1, BLK, H), lambda t, b: (t, b, 0)),
            pl.BlockSpec((1, 1, H), lambda t, b: (t, 0, 0)),
            pl.BlockSpec((1, 1, H), lambda t, b: (t, 0, 0)),
            pl.BlockSpec((1, H), lambda t, b: (0, 0)),
            pl.BlockSpec((1, H), lambda t, b: (0, 0)),
        ],
        out_specs=pl.BlockSpec((1, BLK, H), lambda t, b: (t, b, 0)),
        out_shape=jax.ShapeDtypeStruct((TP, N, H), jnp.float32),
    )(z1, mu, var, gamma, beta)


def _tc_stage_d(Ap, x0, w1t):
    def body(a_ref, x_ref, w_ref, o_ref):
        out2 = 0.9 * a_ref[0, 0] + 0.1 * x_ref[0]
        o_ref[0] = (1.0 - BETA2) * out2 + BETA2 * jnp.dot(
            out2, w_ref[0], preferred_element_type=jnp.float32)

    return pl.pallas_call(
        body,
        grid=(TP, NB),
        in_specs=[
            pl.BlockSpec((1, 1, BLK, H), lambda t, b: (0, t, b, 0)),
            pl.BlockSpec((1, BLK, H), lambda t, b: (t, b, 0)),
            pl.BlockSpec((1, H, H), lambda t, b: (t, 0, 0)),
        ],
        out_specs=pl.BlockSpec((1, BLK, H), lambda t, b: (t, b, 0)),
        out_shape=jax.ShapeDtypeStruct((TP, N, H), jnp.float32),
    )(Ap, x0, w1t)


def _tc_lstm(z2, WihT, WhhT, bsum):
    def body(za_ref, zb_ref, wi_ref, wh_ref, b_ref, o_ref):
        wi = wi_ref[...]
        wh = wh_ref[...]
        bs = b_ref[...]
        g = jnp.dot(za_ref[0], wi, preferred_element_type=jnp.float32) + bs
        i = jax.nn.sigmoid(g[:, 0:H])
        gg = jnp.tanh(g[:, 2 * H:3 * H])
        o = jax.nn.sigmoid(g[:, 3 * H:4 * H])
        c = i * gg
        h = o * jnp.tanh(c)
        g = (jnp.dot(zb_ref[0], wi, preferred_element_type=jnp.float32)
             + jnp.dot(h, wh, preferred_element_type=jnp.float32) + bs)
        i = jax.nn.sigmoid(g[:, 0:H])
        f = jax.nn.sigmoid(g[:, H:2 * H])
        gg = jnp.tanh(g[:, 2 * H:3 * H])
        o = jax.nn.sigmoid(g[:, 3 * H:4 * H])
        c = f * c + i * gg
        h = o * jnp.tanh(c)
        g = jnp.dot(h, wi, preferred_element_type=jnp.float32) + bs
        i = jax.nn.sigmoid(g[:, 0:H])
        f = jax.nn.sigmoid(g[:, H:2 * H])
        gg = jnp.tanh(g[:, 2 * H:3 * H])
        o = jax.nn.sigmoid(g[:, 3 * H:4 * H])
        c = f * c + i * gg
        o_ref[...] = o * jnp.tanh(c)

    return pl.pallas_call(
        body,
        grid=(NB,),
        in_specs=[
            pl.BlockSpec((1, BLK, H), lambda b: (0, b, 0)),
            pl.BlockSpec((1, BLK, H), lambda b: (1, b, 0)),
            pl.BlockSpec((H, 4 * H), lambda b: (0, 0)),
            pl.BlockSpec((H, 4 * H), lambda b: (0, 0)),
            pl.BlockSpec((1, 4 * H), lambda b: (0, 0)),
        ],
        out_specs=pl.BlockSpec((BLK, H), lambda b: (b, 0)),
        out_shape=jax.ShapeDtypeStruct((N, H), jnp.float32),
    )(z2, z2, WihT, WhhT, bsum)


def kernel(x_seq, edge_index_seq, W_gcn, b_gcn, w1_0, w1_1,
           r0_Wih, r0_Whh, r0_bih, r0_bhh,
           r1_Wih, r1_Whh, r1_bih, r1_bhh,
           r2_Wih, r2_Whh, r2_bih, r2_bhh,
           f_Wih, f_Whh, f_bih, f_bhh, bn_gamma, bn_beta):
    f32 = jnp.float32
    xs = x_seq[:TP]
    src = edge_index_seq[:TP, 0]
    dst = edge_index_seq[:TP, 1]

    pad = EP - E
    toff = (jnp.arange(TP, dtype=jnp.int32) * N)[:, None]
    src_g = (jnp.pad(src, ((0, 0), (0, pad))) + toff).reshape(TP, NCH, C)
    pad_ids = N + (jnp.arange(pad, dtype=jnp.int32) % (NP - N))
    dst_p = jnp.concatenate(
        [dst, jnp.broadcast_to(pad_ids, (TP, pad))], axis=1)
    dst_g = dst_p.reshape(TP, NCH, C)
    doff = (jnp.arange(TP, dtype=jnp.int32) * NP)[:, None]
    dst_deg = (dst_p + doff).reshape(TP, NCH, C)

    ones_c = jnp.ones((C,), f32)
    zeros_deg = jnp.zeros((DEGT,), f32)
    zeros_rows = jnp.zeros((ZROWS, H), f32)

    degp = _sc_degrees(dst_deg, ones_c, zeros_deg)
    deg = (degp[0, 0] + degp[1, 0])[:TP * NP].reshape(TP, NP)[:, :N] + 1.0
    dinv3 = lax.rsqrt(deg)[:, :, None]

    w1s = _tc_weights(w1_0, w1_1, jnp.transpose(r1_Wih),
                      jnp.transpose(r2_Wih),
                      (r1_bih + r1_bhh)[None, :], (r2_bih + r2_bhh)[None, :])

    hs = _tc_stage_a(xs, W_gcn, dinv3)
    Sp = _sc_segsum(hs.reshape(TP * N, H), src_g, dst_g, zeros_rows)
    z = _tc_stage_b(Sp, hs, dinv3, b_gcn[None, :])

    A1p = _sc_segsum(z.reshape(TP * N, H), src_g, dst_g, zeros_rows)
    z1, ps, pss = _tc_stage_c1(A1p, z, w1s[0])
    mu = ps.sum(axis=1) / N
    var = pss.sum(axis=1) / N - mu * mu
    z1n = _tc_stage_c2(z1, mu, var, bn_gamma[None, :], bn_beta[None, :])

    A2p = _sc_segsum(z1n.reshape(TP * N, H), src_g, dst_g, zeros_rows)
    z2 = _tc_stage_d(A2p, z, w1s[1])

    h2 = _tc_lstm(z2, jnp.transpose(f_Wih), jnp.transpose(f_Whh),
                  (f_bih + f_bhh)[None, :])

    return jnp.concatenate([z2[:2], h2[None]], axis=0)


# speedup vs baseline: 1.2118x; 1.2118x over previous
import functools

import jax
import jax.numpy as jnp
import numpy as np
from jax import lax
from jax.experimental import pallas as pl
from jax.experimental.pallas import tpu as pltpu
from jax.experimental.pallas import tpu_sc as plsc

N = 10000
E = 320000
H = 128
TP = 3
C = 128
NCH = 2560
EP = NCH * C
CPT0 = 128
CPT1 = 32
NCH0 = CPT0 * 16
HCP0 = CPT0 // 2
HCP1 = CPT1 // 2
NP = 10112
ZROWS = NP // 16
DEGL = 32768
DEGT = DEGL // 16

BLK = 1000
NB = N // BLK

BETA1 = float(np.log(0.5 / 1.0 + 1.0))
BETA2 = float(np.log(0.5 / 2.0 + 1.0))

_MESH = plsc.VectorSubcoreMesh(core_axis_name="c", subcore_axis_name="s")


def _sc_degrees(dst_deg, ones_c, zeros_deg):
    @functools.partial(
        pl.kernel,
        out_type=jax.ShapeDtypeStruct((2, 1, DEGL), jnp.float32),
        mesh=_MESH,
        scratch_types=[
            pltpu.VMEM((NCH // 32, C), jnp.int32),
            pltpu.VMEM((C,), jnp.float32),
            pltpu.VMEM_SHARED((DEGL,), jnp.float32),
            pltpu.SemaphoreType.DMA,
        ],
    )
    def k(dst_hbm, ones_hbm, zero_hbm, out_hbm, idx_v, ones_v, acc, sem):
        cid = lax.axis_index("c")
        sid = lax.axis_index("s")
        wid = sid * 2 + cid
        cpt = NCH // 32
        pltpu.sync_copy(ones_hbm, ones_v)
        pltpu.sync_copy(zero_hbm, acc.at[pl.ds(sid * DEGT, DEGT)])
        plsc.subcore_barrier()
        for t in range(TP):
            pltpu.sync_copy(dst_hbm.at[t, pl.ds(wid * cpt, cpt)], idx_v)

            def body(g, carry):
                pltpu.sync_copy(ones_v, acc.at[idx_v.at[g]], add=True)
                return carry

            lax.fori_loop(0, cpt, body, 0)
        plsc.subcore_barrier()
        pltpu.sync_copy(acc.at[pl.ds(sid * DEGT, DEGT)],
                        out_hbm.at[cid, 0, pl.ds(sid * DEGT, DEGT)])

    return k(dst_deg, ones_c, zeros_deg)


def _sc_segsum(vals_flat, src_g, dst_g, zeros_rows):
    @functools.partial(
        pl.kernel,
        out_type=jax.ShapeDtypeStruct((2, TP, NP, H), jnp.float32),
        mesh=_MESH,
        scratch_types=[
            pltpu.VMEM((HCP0, C), jnp.int32),
            pltpu.VMEM((HCP0, C), jnp.int32),
            pltpu.VMEM((C, H), jnp.float32),
            pltpu.VMEM((C, H), jnp.float32),
            pltpu.VMEM_SHARED((NP, H), jnp.float32),
            pltpu.SemaphoreType.DMA,
            pltpu.SemaphoreType.DMA,
        ],
    )
    def k(vals_hbm, src_hbm, dst_hbm, zero_hbm, out_hbm,
          src_v, dst_v, rows0, rows1, acc, sem0, sem1):
        cid = lax.axis_index("c")
        sid = lax.axis_index("s")

        def run_half(t, base, hcp):
            pltpu.sync_copy(src_hbm.at[t, pl.ds(base, hcp)],
                            src_v.at[pl.ds(0, hcp)])
            pltpu.sync_copy(dst_hbm.at[t, pl.ds(base, hcp)],
                            dst_v.at[pl.ds(0, hcp)])
            pltpu.async_copy(vals_hbm.at[src_v.at[0]], rows0, sem0)

            def body(gp, carry):
                g0 = 2 * gp
                pltpu.async_copy(vals_hbm.at[src_v.at[g0 + 1]], rows1, sem1)
                pltpu.make_async_copy(vals_hbm.at[src_v.at[g0]], rows0,
                                      sem0).wait()
                pltpu.sync_copy(rows0, acc.at[dst_v.at[g0]], add=True)

                @pl.when(gp + 1 < hcp // 2)
                def _():
                    pltpu.async_copy(vals_hbm.at[src_v.at[g0 + 2]], rows0,
                                     sem0)

                pltpu.make_async_copy(vals_hbm.at[src_v.at[g0 + 1]], rows1,
                                      sem1).wait()
                pltpu.sync_copy(rows1, acc.at[dst_v.at[g0 + 1]], add=True)
                return carry

            lax.fori_loop(0, hcp // 2, body, 0)

        for t in range(TP):
            pltpu.sync_copy(zero_hbm, acc.at[pl.ds(sid * ZROWS, ZROWS)])
            plsc.subcore_barrier()

            @pl.when(cid == 0)
            def _():
                for hp in range(2):
                    run_half(t, sid * CPT0 + hp * HCP0, HCP0)

            @pl.when(cid == 1)
            def _():
                for hp in range(2):
                    run_half(t, NCH0 + sid * CPT1 + hp * HCP1, HCP1)

            plsc.subcore_barrier()
            pltpu.sync_copy(acc.at[pl.ds(sid * ZROWS, ZROWS)],
                            out_hbm.at[cid, t, pl.ds(sid * ZROWS, ZROWS)])
            plsc.subcore_barrier()

    return k(vals_flat, src_g, dst_g, zeros_rows)


def _tc_weights(w1_0, w1_1, r1_WihT, r2_WihT, b1sum, b2sum):
    def body(w0_ref, w1_ref, wt1_ref, wt2_ref, b1_ref, b2_ref, o_ref):
        for li, (w0, wt, bs) in enumerate(
                ((w0_ref[...], wt1_ref, b1_ref),
                 (w1_ref[...], wt2_ref, b2_ref))):
            w = w0
            c = jnp.zeros((H, H), jnp.float32)
            for t in range(TP):
                g = jnp.dot(w, wt[...], preferred_element_type=jnp.float32) + bs[...]
                i = jax.nn.sigmoid(g[:, 0:H])
                f = jax.nn.sigmoid(g[:, H:2 * H])
                gg = jnp.tanh(g[:, 2 * H:3 * H])
                o = jax.nn.sigmoid(g[:, 3 * H:4 * H])
                c = f * c + i * gg
                w = o * jnp.tanh(c)
                o_ref[li, t] = w

    return pl.pallas_call(
        body,
        out_shape=jax.ShapeDtypeStruct((2, TP, H, H), jnp.float32),
    )(w1_0, w1_1, r1_WihT, r2_WihT, b1sum, b2sum)


def _tc_stage_a(x3, Wg, dinv3):
    def body(x_ref, w_ref, d_ref, o_ref):
        o_ref[0] = jnp.dot(x_ref[0], w_ref[...],
                           preferred_element_type=jnp.float32) * d_ref[0]

    return pl.pallas_call(
        body,
        grid=(TP, NB),
        in_specs=[
            pl.BlockSpec((1, BLK, H), lambda t, b: (t, b, 0)),
            pl.BlockSpec((H, H), lambda t, b: (0, 0)),
            pl.BlockSpec((1, BLK, 1), lambda t, b: (t, b, 0)),
        ],
        out_specs=pl.BlockSpec((1, BLK, H), lambda t, b: (t, b, 0)),
        out_shape=jax.ShapeDtypeStruct((TP, N, H), jnp.float32),
    )(x3, Wg, dinv3)


def _tc_stage_b(Sp, hs, dinv3, bias):
    def body(s_ref, h_ref, d_ref, b_ref, o_ref):
        o_ref[0] = (s_ref[0, 0] + s_ref[1, 0] + h_ref[0]) * d_ref[0] + b_ref[...]

    return pl.pallas_call(
        body,
        grid=(TP, NB),
        in_specs=[
            pl.BlockSpec((2, 1, BLK, H), lambda t, b: (0, t, b, 0)),
            pl.BlockSpec((1, BLK, H), lambda t, b: (t, b, 0)),
            pl.BlockSpec((1, BLK, 1), lambda t, b: (t, b, 0)),
            pl.BlockSpec((1, H), lambda t, b: (0, 0)),
        ],
        out_specs=pl.BlockSpec((1, BLK, H), lambda t, b: (t, b, 0)),
        out_shape=jax.ShapeDtypeStruct((TP, N, H), jnp.float32),
    )(Sp, hs, dinv3, bias)


def _tc_stage_c1(Ap, z, w1t):
    def body(a_ref, z_ref, w_ref, o_ref, ps_ref, pss_ref):
        zv = z_ref[0]
        out1 = 0.9 * (a_ref[0, 0] + a_ref[1, 0]) + 0.1 * zv
        z1 = (1.0 - BETA1) * out1 + BETA1 * jnp.dot(
            out1, w_ref[0], preferred_element_type=jnp.float32)
        o_ref[0] = z1
        ps_ref[0, 0, 0] = jnp.sum(z1, axis=0)
        pss_ref[0, 0, 0] = jnp.sum(z1 * z1, axis=0)

    return pl.pallas_call(
        body,
        grid=(TP, NB),
        in_specs=[
            pl.BlockSpec((2, 1, BLK, H), lambda t, b: (0, t, b, 0)),
            pl.BlockSpec((1, BLK, H), lambda t, b: (t, b, 0)),
            pl.BlockSpec((1, H, H), lambda t, b: (t, 0, 0)),
        ],
        out_specs=[
            pl.BlockSpec((1, BLK, H), lambda t, b: (t, b, 0)),
            pl.BlockSpec((1, 1, 1, H), lambda t, b: (t, b, 0, 0)),
            pl.BlockSpec((1, 1, 1, H), lambda t, b: (t, b, 0, 0)),
        ],
        out_shape=[
            jax.ShapeDtypeStruct((TP, N, H), jnp.float32),
            jax.ShapeDtypeStruct((TP, NB, 1, H), jnp.float32),
            jax.ShapeDtypeStruct((TP, NB, 1, H), jnp.float32),
        ],
    )(Ap, z, w1t)


def _tc_stage_c2(z1, mu, var, gamma, beta):
    def body(z_ref, m_ref, v_ref, g_ref, b_ref, o_ref):
        scale = lax.rsqrt(v_ref[0] + 1e-5) * g_ref[...]
        o_ref[0] = jax.nn.relu((z_ref[0] - m_ref[0]) * scale + b_ref[...])

    return pl.pallas_call(
        body,
        grid=(TP, NB),
        in_specs=[
            pl.BlockSpec((1, BLK, H), lambda t, b: (t, b, 0)),
            pl.BlockSpec((1, 1, H), lambda t, b: (t, 0, 0)),
            pl.BlockSpec((1, 1, H), lambda t, b: (t, 0, 0)),
            pl.BlockSpec((1, H), lambda t, b: (0, 0)),
            pl.BlockSpec((1, H), lambda t, b: (0, 0)),
        ],
        out_specs=pl.BlockSpec((1, BLK, H), lambda t, b: (t, b, 0)),
        out_shape=jax.ShapeDtypeStruct((TP, N, H), jnp.float32),
    )(z1, mu, var, gamma, beta)


def _tc_stage_d(Ap, x0, w1t):
    def body(a_ref, x_ref, w_ref, o_ref):
        out2 = 0.9 * (a_ref[0, 0] + a_ref[1, 0]) + 0.1 * x_ref[0]
        o_ref[0] = (1.0 - BETA2) * out2 + BETA2 * jnp.dot(
            out2, w_ref[0], preferred_element_type=jnp.float32)

    return pl.pallas_call(
        body,
        grid=(TP, NB),
        in_specs=[
            pl.BlockSpec((2, 1, BLK, H), lambda t, b: (0, t, b, 0)),
            pl.BlockSpec((1, BLK, H), lambda t, b: (t, b, 0)),
            pl.BlockSpec((1, H, H), lambda t, b: (t, 0, 0)),
        ],
        out_specs=pl.BlockSpec((1, BLK, H), lambda t, b: (t, b, 0)),
        out_shape=jax.ShapeDtypeStruct((TP, N, H), jnp.float32),
    )(Ap, x0, w1t)


def _tc_lstm(z2, WihT, WhhT, bsum):
    def body(za_ref, zb_ref, wi_ref, wh_ref, b_ref, o_ref):
        wi = wi_ref[...]
        wh = wh_ref[...]
        bs = b_ref[...]
        g = jnp.dot(za_ref[0], wi, preferred_element_type=jnp.float32) + bs
        i = jax.nn.sigmoid(g[:, 0:H])
        gg = jnp.tanh(g[:, 2 * H:3 * H])
        o = jax.nn.sigmoid(g[:, 3 * H:4 * H])
        c = i * gg
        h = o * jnp.tanh(c)
        g = (jnp.dot(zb_ref[0], wi, preferred_element_type=jnp.float32)
             + jnp.dot(h, wh, preferred_element_type=jnp.float32) + bs)
        i = jax.nn.sigmoid(g[:, 0:H])
        f = jax.nn.sigmoid(g[:, H:2 * H])
        gg = jnp.tanh(g[:, 2 * H:3 * H])
        o = jax.nn.sigmoid(g[:, 3 * H:4 * H])
        c = f * c + i * gg
        h = o * jnp.tanh(c)
        g = jnp.dot(h, wi, preferred_element_type=jnp.float32) + bs
        i = jax.nn.sigmoid(g[:, 0:H])
        f = jax.nn.sigmoid(g[:, H:2 * H])
        gg = jnp.tanh(g[:, 2 * H:3 * H])
        o = jax.nn.sigmoid(g[:, 3 * H:4 * H])
        c = f * c + i * gg
        o_ref[...] = o * jnp.tanh(c)

    return pl.pallas_call(
        body,
        grid=(NB,),
        in_specs=[
            pl.BlockSpec((1, BLK, H), lambda b: (0, b, 0)),
            pl.BlockSpec((1, BLK, H), lambda b: (1, b, 0)),
            pl.BlockSpec((H, 4 * H), lambda b: (0, 0)),
            pl.BlockSpec((H, 4 * H), lambda b: (0, 0)),
            pl.BlockSpec((1, 4 * H), lambda b: (0, 0)),
        ],
        out_specs=pl.BlockSpec((BLK, H), lambda b: (b, 0)),
        out_shape=jax.ShapeDtypeStruct((N, H), jnp.float32),
    )(z2, z2, WihT, WhhT, bsum)


def kernel(x_seq, edge_index_seq, W_gcn, b_gcn, w1_0, w1_1,
           r0_Wih, r0_Whh, r0_bih, r0_bhh,
           r1_Wih, r1_Whh, r1_bih, r1_bhh,
           r2_Wih, r2_Whh, r2_bih, r2_bhh,
           f_Wih, f_Whh, f_bih, f_bhh, bn_gamma, bn_beta):
    f32 = jnp.float32
    xs = x_seq[:TP]
    src = edge_index_seq[:TP, 0]
    dst = edge_index_seq[:TP, 1]

    pad = EP - E
    toff = (jnp.arange(TP, dtype=jnp.int32) * N)[:, None]
    src_g = (jnp.pad(src, ((0, 0), (0, pad))) + toff).reshape(TP, NCH, C)
    pad_ids = N + (jnp.arange(pad, dtype=jnp.int32) % (NP - N))
    dst_p = jnp.concatenate(
        [dst, jnp.broadcast_to(pad_ids, (TP, pad))], axis=1)
    dst_g = dst_p.reshape(TP, NCH, C)
    doff = (jnp.arange(TP, dtype=jnp.int32) * NP)[:, None]
    dst_deg = (dst_p + doff).reshape(TP, NCH, C)

    ones_c = jnp.ones((C,), f32)
    zeros_deg = jnp.zeros((DEGT,), f32)
    zeros_rows = jnp.zeros((ZROWS, H), f32)

    degp = _sc_degrees(dst_deg, ones_c, zeros_deg)
    deg = (degp[0, 0] + degp[1, 0])[:TP * NP].reshape(TP, NP)[:, :N] + 1.0
    dinv3 = lax.rsqrt(deg)[:, :, None]

    w1s = _tc_weights(w1_0, w1_1, jnp.transpose(r1_Wih),
                      jnp.transpose(r2_Wih),
                      (r1_bih + r1_bhh)[None, :], (r2_bih + r2_bhh)[None, :])

    hs = _tc_stage_a(xs, W_gcn, dinv3)
    Sp = _sc_segsum(hs.reshape(TP * N, H), src_g, dst_g, zeros_rows)
    z = _tc_stage_b(Sp, hs, dinv3, b_gcn[None, :])

    A1p = _sc_segsum(z.reshape(TP * N, H), src_g, dst_g, zeros_rows)
    z1, ps, pss = _tc_stage_c1(A1p, z, w1s[0])
    mu = ps.sum(axis=1) / N
    var = pss.sum(axis=1) / N - mu * mu
    z1n = _tc_stage_c2(z1, mu, var, bn_gamma[None, :], bn_beta[None, :])

    A2p = _sc_segsum(z1n.reshape(TP * N, H), src_g, dst_g, zeros_rows)
    z2 = _tc_stage_d(A2p, z, w1s[1])

    h2 = _tc_lstm(z2, jnp.transpose(f_Wih), jnp.transpose(f_Whh),
                  (f_bih + f_bhh)[None, :])

    return jnp.concatenate([z2[:2], h2[None]], axis=0)


# speedup vs baseline: 1.3645x; 1.1260x over previous
import functools

import jax
import jax.numpy as jnp
import numpy as np
from jax import lax
from jax.experimental import pallas as pl
from jax.experimental.pallas import tpu as pltpu
from jax.experimental.pallas import tpu_sc as plsc

N = 10000
E = 320000
H = 128
TP = 3
C = 128
NCH = 2560
EP = NCH * C
CPT0 = 144
CPT1 = 16
NCH0 = CPT0 * 16
HCP0 = 64
HCP1 = CPT1 // 2
NP = 10112
ZROWS = NP // 16
DEGL = 32768
DEGT = DEGL // 16

BLK = 1000
NB = N // BLK

BETA1 = float(np.log(0.5 / 1.0 + 1.0))
BETA2 = float(np.log(0.5 / 2.0 + 1.0))

_MESH = plsc.VectorSubcoreMesh(core_axis_name="c", subcore_axis_name="s")


def _sc_degrees(dst_deg, ones_c, zeros_deg):
    @functools.partial(
        pl.kernel,
        out_type=jax.ShapeDtypeStruct((2, 1, DEGL), jnp.float32),
        mesh=_MESH,
        scratch_types=[
            pltpu.VMEM((NCH // 32, C), jnp.int32),
            pltpu.VMEM((C,), jnp.float32),
            pltpu.VMEM_SHARED((DEGL,), jnp.float32),
            pltpu.SemaphoreType.DMA,
        ],
    )
    def k(dst_hbm, ones_hbm, zero_hbm, out_hbm, idx_v, ones_v, acc, sem):
        cid = lax.axis_index("c")
        sid = lax.axis_index("s")
        wid = sid * 2 + cid
        cpt = NCH // 32
        pltpu.sync_copy(ones_hbm, ones_v)
        pltpu.sync_copy(zero_hbm, acc.at[pl.ds(sid * DEGT, DEGT)])
        plsc.subcore_barrier()
        for t in range(TP):
            pltpu.sync_copy(dst_hbm.at[t, pl.ds(wid * cpt, cpt)], idx_v)

            def body(g, carry):
                pltpu.sync_copy(ones_v, acc.at[idx_v.at[g]], add=True)
                return carry

            lax.fori_loop(0, cpt, body, 0)
        plsc.subcore_barrier()
        pltpu.sync_copy(acc.at[pl.ds(sid * DEGT, DEGT)],
                        out_hbm.at[cid, 0, pl.ds(sid * DEGT, DEGT)])

    return k(dst_deg, ones_c, zeros_deg)


def _sc_segsum(vals_flat, src_g, dst_g, zeros_rows):
    @functools.partial(
        pl.kernel,
        out_type=jax.ShapeDtypeStruct((2, TP, NP, H), jnp.float32),
        mesh=_MESH,
        scratch_types=[
            pltpu.VMEM((HCP0, C), jnp.int32),
            pltpu.VMEM((HCP0, C), jnp.int32),
            pltpu.VMEM((C, H), jnp.float32),
            pltpu.VMEM((C, H), jnp.float32),
            pltpu.VMEM_SHARED((NP, H), jnp.float32),
            pltpu.SemaphoreType.DMA,
            pltpu.SemaphoreType.DMA,
        ],
    )
    def k(vals_hbm, src_hbm, dst_hbm, zero_hbm, out_hbm,
          src_v, dst_v, rows0, rows1, acc, sem0, sem1):
        cid = lax.axis_index("c")
        sid = lax.axis_index("s")

        def run_half(t, base, hcp):
            pltpu.sync_copy(src_hbm.at[t, pl.ds(base, hcp)],
                            src_v.at[pl.ds(0, hcp)])
            pltpu.sync_copy(dst_hbm.at[t, pl.ds(base, hcp)],
                            dst_v.at[pl.ds(0, hcp)])
            pltpu.async_copy(vals_hbm.at[src_v.at[0]], rows0, sem0)

            def body(gp, carry):
                g0 = 2 * gp
                pltpu.async_copy(vals_hbm.at[src_v.at[g0 + 1]], rows1, sem1)
                pltpu.make_async_copy(vals_hbm.at[src_v.at[g0]], rows0,
                                      sem0).wait()
                pltpu.sync_copy(rows0, acc.at[dst_v.at[g0]], add=True)

                @pl.when(gp + 1 < hcp // 2)
                def _():
                    pltpu.async_copy(vals_hbm.at[src_v.at[g0 + 2]], rows0,
                                     sem0)

                pltpu.make_async_copy(vals_hbm.at[src_v.at[g0 + 1]], rows1,
                                      sem1).wait()
                pltpu.sync_copy(rows1, acc.at[dst_v.at[g0 + 1]], add=True)
                return carry

            lax.fori_loop(0, hcp // 2, body, 0)

        for t in range(TP):
            pltpu.sync_copy(zero_hbm, acc.at[pl.ds(sid * ZROWS, ZROWS)])
            plsc.subcore_barrier()

            @pl.when(cid == 0)
            def _():
                base0 = sid * CPT0
                run_half(t, base0, HCP0)
                run_half(t, base0 + HCP0, HCP0)
                run_half(t, base0 + 2 * HCP0, CPT0 - 2 * HCP0)

            @pl.when(cid == 1)
            def _():
                for hp in range(2):
                    run_half(t, NCH0 + sid * CPT1 + hp * HCP1, HCP1)

            plsc.subcore_barrier()
            pltpu.sync_copy(acc.at[pl.ds(sid * ZROWS, ZROWS)],
                            out_hbm.at[cid, t, pl.ds(sid * ZROWS, ZROWS)])
            plsc.subcore_barrier()

    return k(vals_flat, src_g, dst_g, zeros_rows)


def _tc_weights(w1_0, w1_1, r1_WihT, r2_WihT, b1sum, b2sum):
    def body(w0_ref, w1_ref, wt1_ref, wt2_ref, b1_ref, b2_ref, o_ref):
        for li, (w0, wt, bs) in enumerate(
                ((w0_ref[...], wt1_ref, b1_ref),
                 (w1_ref[...], wt2_ref, b2_ref))):
            w = w0
            c = jnp.zeros((H, H), jnp.float32)
            for t in range(TP):
                g = jnp.dot(w, wt[...], preferred_element_type=jnp.float32) + bs[...]
                i = jax.nn.sigmoid(g[:, 0:H])
                f = jax.nn.sigmoid(g[:, H:2 * H])
                gg = jnp.tanh(g[:, 2 * H:3 * H])
                o = jax.nn.sigmoid(g[:, 3 * H:4 * H])
                c = f * c + i * gg
                w = o * jnp.tanh(c)
                o_ref[li, t] = w

    return pl.pallas_call(
        body,
        out_shape=jax.ShapeDtypeStruct((2, TP, H, H), jnp.float32),
    )(w1_0, w1_1, r1_WihT, r2_WihT, b1sum, b2sum)


def _tc_stage_a(x3, Wg, dinv3):
    def body(x_ref, w_ref, d_ref, o_ref):
        o_ref[0] = jnp.dot(x_ref[0], w_ref[...],
                           preferred_element_type=jnp.float32) * d_ref[0]

    return pl.pallas_call(
        body,
        grid=(TP, NB),
        in_specs=[
            pl.BlockSpec((1, BLK, H), lambda t, b: (t, b, 0)),
            pl.BlockSpec((H, H), lambda t, b: (0, 0)),
            pl.BlockSpec((1, BLK, 1), lambda t, b: (t, b, 0)),
        ],
        out_specs=pl.BlockSpec((1, BLK, H), lambda t, b: (t, b, 0)),
        out_shape=jax.ShapeDtypeStruct((TP, N, H), jnp.float32),
    )(x3, Wg, dinv3)


def _tc_stage_b(Sp, hs, dinv3, bias):
    def body(s_ref, h_ref, d_ref, b_ref, o_ref):
        o_ref[0] = (s_ref[0, 0] + s_ref[1, 0] + h_ref[0]) * d_ref[0] + b_ref[...]

    return pl.pallas_call(
        body,
        grid=(TP, NB),
        in_specs=[
            pl.BlockSpec((2, 1, BLK, H), lambda t, b: (0, t, b, 0)),
            pl.BlockSpec((1, BLK, H), lambda t, b: (t, b, 0)),
            pl.BlockSpec((1, BLK, 1), lambda t, b: (t, b, 0)),
            pl.BlockSpec((1, H), lambda t, b: (0, 0)),
        ],
        out_specs=pl.BlockSpec((1, BLK, H), lambda t, b: (t, b, 0)),
        out_shape=jax.ShapeDtypeStruct((TP, N, H), jnp.float32),
    )(Sp, hs, dinv3, bias)


def _tc_stage_c1(Ap, z, w1t):
    def body(a_ref, z_ref, w_ref, o_ref, ps_ref, pss_ref):
        zv = z_ref[0]
        out1 = 0.9 * (a_ref[0, 0] + a_ref[1, 0]) + 0.1 * zv
        z1 = (1.0 - BETA1) * out1 + BETA1 * jnp.dot(
            out1, w_ref[0], preferred_element_type=jnp.float32)
        o_ref[0] = z1
        ps_ref[0, 0, 0] = jnp.sum(z1, axis=0)
        pss_ref[0, 0, 0] = jnp.sum(z1 * z1, axis=0)

    return pl.pallas_call(
        body,
        grid=(TP, NB),
        in_specs=[
            pl.BlockSpec((2, 1, BLK, H), lambda t, b: (0, t, b, 0)),
            pl.BlockSpec((1, BLK, H), lambda t, b: (t, b, 0)),
            pl.BlockSpec((1, H, H), lambda t, b: (t, 0, 0)),
        ],
        out_specs=[
            pl.BlockSpec((1, BLK, H), lambda t, b: (t, b, 0)),
            pl.BlockSpec((1, 1, 1, H), lambda t, b: (t, b, 0, 0)),
            pl.BlockSpec((1, 1, 1, H), lambda t, b: (t, b, 0, 0)),
        ],
        out_shape=[
            jax.ShapeDtypeStruct((TP, N, H), jnp.float32),
            jax.ShapeDtypeStruct((TP, NB, 1, H), jnp.float32),
            jax.ShapeDtypeStruct((TP, NB, 1, H), jnp.float32),
        ],
    )(Ap, z, w1t)


def _tc_stage_c2(z1, mu, var, gamma, beta):
    def body(z_ref, m_ref, v_ref, g_ref, b_ref, o_ref):
        scale = lax.rsqrt(v_ref[0] + 1e-5) * g_ref[...]
        o_ref[0] = jax.nn.relu((z_ref[0] - m_ref[0]) * scale + b_ref[...])

    return pl.pallas_call(
        body,
        grid=(TP, NB),
        in_specs=[
            pl.BlockSpec((1, BLK, H), lambda t, b: (t, b, 0)),
            pl.BlockSpec((1, 1, H), lambda t, b: (t, 0, 0)),
            pl.BlockSpec((1, 1, H), lambda t, b: (t, 0, 0)),
            pl.BlockSpec((1, H), lambda t, b: (0, 0)),
            pl.BlockSpec((1, H), lambda t, b: (0, 0)),
        ],
        out_specs=pl.BlockSpec((1, BLK, H), lambda t, b: (t, b, 0)),
        out_shape=jax.ShapeDtypeStruct((TP, N, H), jnp.float32),
    )(z1, mu, var, gamma, beta)


def _tc_stage_d(Ap, x0, w1t):
    def body(a_ref, x_ref, w_ref, o_ref):
        out2 = 0.9 * (a_ref[0, 0] + a_ref[1, 0]) + 0.1 * x_ref[0]
        o_ref[0] = (1.0 - BETA2) * out2 + BETA2 * jnp.dot(
            out2, w_ref[0], preferred_element_type=jnp.float32)

    return pl.pallas_call(
        body,
        grid=(TP, NB),
        in_specs=[
            pl.BlockSpec((2, 1, BLK, H), lambda t, b: (0, t, b, 0)),
            pl.BlockSpec((1, BLK, H), lambda t, b: (t, b, 0)),
            pl.BlockSpec((1, H, H), lambda t, b: (t, 0, 0)),
        ],
        out_specs=pl.BlockSpec((1, BLK, H), lambda t, b: (t, b, 0)),
        out_shape=jax.ShapeDtypeStruct((TP, N, H), jnp.float32),
    )(Ap, x0, w1t)


def _tc_lstm(z2, WihT, WhhT, bsum):
    def body(za_ref, zb_ref, wi_ref, wh_ref, b_ref, o_ref):
        wi = wi_ref[...]
        wh = wh_ref[...]
        bs = b_ref[...]
        g = jnp.dot(za_ref[0], wi, preferred_element_type=jnp.float32) + bs
        i = jax.nn.sigmoid(g[:, 0:H])
        gg = jnp.tanh(g[:, 2 * H:3 * H])
        o = jax.nn.sigmoid(g[:, 3 * H:4 * H])
        c = i * gg
        h = o * jnp.tanh(c)
        g = (jnp.dot(zb_ref[0], wi, preferred_element_type=jnp.float32)
             + jnp.dot(h, wh, preferred_element_type=jnp.float32) + bs)
        i = jax.nn.sigmoid(g[:, 0:H])
        f = jax.nn.sigmoid(g[:, H:2 * H])
        gg = jnp.tanh(g[:, 2 * H:3 * H])
        o = jax.nn.sigmoid(g[:, 3 * H:4 * H])
        c = f * c + i * gg
        h = o * jnp.tanh(c)
        g = jnp.dot(h, wi, preferred_element_type=jnp.float32) + bs
        i = jax.nn.sigmoid(g[:, 0:H])
        f = jax.nn.sigmoid(g[:, H:2 * H])
        gg = jnp.tanh(g[:, 2 * H:3 * H])
        o = jax.nn.sigmoid(g[:, 3 * H:4 * H])
        c = f * c + i * gg
        o_ref[...] = o * jnp.tanh(c)

    return pl.pallas_call(
        body,
        grid=(NB,),
        in_specs=[
            pl.BlockSpec((1, BLK, H), lambda b: (0, b, 0)),
            pl.BlockSpec((1, BLK, H), lambda b: (1, b, 0)),
            pl.BlockSpec((H, 4 * H), lambda b: (0, 0)),
            pl.BlockSpec((H, 4 * H), lambda b: (0, 0)),
            pl.BlockSpec((1, 4 * H), lambda b: (0, 0)),
        ],
        out_specs=pl.BlockSpec((BLK, H), lambda b: (b, 0)),
        out_shape=jax.ShapeDtypeStruct((N, H), jnp.float32),
    )(z2, z2, WihT, WhhT, bsum)


def kernel(x_seq, edge_index_seq, W_gcn, b_gcn, w1_0, w1_1,
           r0_Wih, r0_Whh, r0_bih, r0_bhh,
           r1_Wih, r1_Whh, r1_bih, r1_bhh,
           r2_Wih, r2_Whh, r2_bih, r2_bhh,
           f_Wih, f_Whh, f_bih, f_bhh, bn_gamma, bn_beta):
    f32 = jnp.float32
    xs = x_seq[:TP]
    src = edge_index_seq[:TP, 0]
    dst = edge_index_seq[:TP, 1]

    pad = EP - E
    toff = (jnp.arange(TP, dtype=jnp.int32) * N)[:, None]
    src_g = (jnp.pad(src, ((0, 0), (0, pad))) + toff).reshape(TP, NCH, C)
    pad_ids = N + (jnp.arange(pad, dtype=jnp.int32) % (NP - N))
    dst_p = jnp.concatenate(
        [dst, jnp.broadcast_to(pad_ids, (TP, pad))], axis=1)
    dst_g = dst_p.reshape(TP, NCH, C)
    doff = (jnp.arange(TP, dtype=jnp.int32) * NP)[:, None]
    dst_deg = (dst_p + doff).reshape(TP, NCH, C)

    ones_c = jnp.ones((C,), f32)
    zeros_deg = jnp.zeros((DEGT,), f32)
    zeros_rows = jnp.zeros((ZROWS, H), f32)

    degp = _sc_degrees(dst_deg, ones_c, zeros_deg)
    deg = (degp[0, 0] + degp[1, 0])[:TP * NP].reshape(TP, NP)[:, :N] + 1.0
    dinv3 = lax.rsqrt(deg)[:, :, None]

    w1s = _tc_weights(w1_0, w1_1, jnp.transpose(r1_Wih),
                      jnp.transpose(r2_Wih),
                      (r1_bih + r1_bhh)[None, :], (r2_bih + r2_bhh)[None, :])

    hs = _tc_stage_a(xs, W_gcn, dinv3)
    Sp = _sc_segsum(hs.reshape(TP * N, H), src_g, dst_g, zeros_rows)
    z = _tc_stage_b(Sp, hs, dinv3, b_gcn[None, :])

    A1p = _sc_segsum(z.reshape(TP * N, H), src_g, dst_g, zeros_rows)
    z1, ps, pss = _tc_stage_c1(A1p, z, w1s[0])
    mu = ps.sum(axis=1) / N
    var = pss.sum(axis=1) / N - mu * mu
    z1n = _tc_stage_c2(z1, mu, var, bn_gamma[None, :], bn_beta[None, :])

    A2p = _sc_segsum(z1n.reshape(TP * N, H), src_g, dst_g, zeros_rows)
    z2 = _tc_stage_d(A2p, z, w1s[1])

    h2 = _tc_lstm(z2, jnp.transpose(f_Wih), jnp.transpose(f_Whh),
                  (f_bih + f_bhh)[None, :])

    return jnp.concatenate([z2[:2], h2[None]], axis=0)


# speedup vs baseline: 1.3802x; 1.0115x over previous
import functools

import jax
import jax.numpy as jnp
import numpy as np
from jax import lax
from jax.experimental import pallas as pl
from jax.experimental.pallas import tpu as pltpu
from jax.experimental.pallas import tpu_sc as plsc

N = 10000
E = 320000
H = 128
TP = 3
C = 128
NCH = 2560
EP = NCH * C
CPT0 = 152
CPT1 = 8
NCH0 = CPT0 * 16
HCP0 = 64
HCP1 = CPT1
NP = 10112
ZROWS = NP // 16
DEGL = 32768
DEGT = DEGL // 16

BLK = 1000
NB = N // BLK

BETA1 = float(np.log(0.5 / 1.0 + 1.0))
BETA2 = float(np.log(0.5 / 2.0 + 1.0))

_MESH = plsc.VectorSubcoreMesh(core_axis_name="c", subcore_axis_name="s")


def _sc_degrees(dst_deg, ones_c, zeros_deg):
    @functools.partial(
        pl.kernel,
        out_type=jax.ShapeDtypeStruct((2, 1, DEGL), jnp.float32),
        mesh=_MESH,
        scratch_types=[
            pltpu.VMEM((NCH // 32, C), jnp.int32),
            pltpu.VMEM((C,), jnp.float32),
            pltpu.VMEM_SHARED((DEGL,), jnp.float32),
            pltpu.SemaphoreType.DMA,
        ],
    )
    def k(dst_hbm, ones_hbm, zero_hbm, out_hbm, idx_v, ones_v, acc, sem):
        cid = lax.axis_index("c")
        sid = lax.axis_index("s")
        wid = sid * 2 + cid
        cpt = NCH // 32
        pltpu.sync_copy(ones_hbm, ones_v)
        pltpu.sync_copy(zero_hbm, acc.at[pl.ds(sid * DEGT, DEGT)])
        plsc.subcore_barrier()
        for t in range(TP):
            pltpu.sync_copy(dst_hbm.at[t, pl.ds(wid * cpt, cpt)], idx_v)

            def body(g, carry):
                pltpu.sync_copy(ones_v, acc.at[idx_v.at[g]], add=True)
                return carry

            lax.fori_loop(0, cpt, body, 0)
        plsc.subcore_barrier()
        pltpu.sync_copy(acc.at[pl.ds(sid * DEGT, DEGT)],
                        out_hbm.at[cid, 0, pl.ds(sid * DEGT, DEGT)])

    return k(dst_deg, ones_c, zeros_deg)


def _sc_segsum(vals_flat, src_g, dst_g, zeros_rows):
    @functools.partial(
        pl.kernel,
        out_type=jax.ShapeDtypeStruct((2, TP, NP, H), jnp.float32),
        mesh=_MESH,
        scratch_types=[
            pltpu.VMEM((HCP0, C), jnp.int32),
            pltpu.VMEM((HCP0, C), jnp.int32),
            pltpu.VMEM((C, H), jnp.float32),
            pltpu.VMEM((C, H), jnp.float32),
            pltpu.VMEM_SHARED((NP, H), jnp.float32),
            pltpu.SemaphoreType.DMA,
            pltpu.SemaphoreType.DMA,
        ],
    )
    def k(vals_hbm, src_hbm, dst_hbm, zero_hbm, out_hbm,
          src_v, dst_v, rows0, rows1, acc, sem0, sem1):
        cid = lax.axis_index("c")
        sid = lax.axis_index("s")

        def run_half(t, base, hcp):
            pltpu.sync_copy(src_hbm.at[t, pl.ds(base, hcp)],
                            src_v.at[pl.ds(0, hcp)])
            pltpu.sync_copy(dst_hbm.at[t, pl.ds(base, hcp)],
                            dst_v.at[pl.ds(0, hcp)])
            pltpu.async_copy(vals_hbm.at[src_v.at[0]], rows0, sem0)

            def body(gp, carry):
                g0 = 2 * gp
                pltpu.async_copy(vals_hbm.at[src_v.at[g0 + 1]], rows1, sem1)
                pltpu.make_async_copy(vals_hbm.at[src_v.at[g0]], rows0,
                                      sem0).wait()
                pltpu.sync_copy(rows0, acc.at[dst_v.at[g0]], add=True)

                @pl.when(gp + 1 < hcp // 2)
                def _():
                    pltpu.async_copy(vals_hbm.at[src_v.at[g0 + 2]], rows0,
                                     sem0)

                pltpu.make_async_copy(vals_hbm.at[src_v.at[g0 + 1]], rows1,
                                      sem1).wait()
                pltpu.sync_copy(rows1, acc.at[dst_v.at[g0 + 1]], add=True)
                return carry

            lax.fori_loop(0, hcp // 2, body, 0)

        for t in range(TP):
            pltpu.sync_copy(zero_hbm, acc.at[pl.ds(sid * ZROWS, ZROWS)])
            plsc.subcore_barrier()

            @pl.when(cid == 0)
            def _():
                base0 = sid * CPT0
                run_half(t, base0, HCP0)
                run_half(t, base0 + HCP0, HCP0)
                run_half(t, base0 + 2 * HCP0, CPT0 - 2 * HCP0)

            @pl.when(cid == 1)
            def _():
                run_half(t, NCH0 + sid * CPT1, HCP1)

            plsc.subcore_barrier()
            pltpu.sync_copy(acc.at[pl.ds(sid * ZROWS, ZROWS)],
                            out_hbm.at[cid, t, pl.ds(sid * ZROWS, ZROWS)])
            plsc.subcore_barrier()

    return k(vals_flat, src_g, dst_g, zeros_rows)


def _tc_weights(w1_0, w1_1, r1_WihT, r2_WihT, b1sum, b2sum):
    def body(w0_ref, w1_ref, wt1_ref, wt2_ref, b1_ref, b2_ref, o_ref):
        for li, (w0, wt, bs) in enumerate(
                ((w0_ref[...], wt1_ref, b1_ref),
                 (w1_ref[...], wt2_ref, b2_ref))):
            w = w0
            c = jnp.zeros((H, H), jnp.float32)
            for t in range(TP):
                g = jnp.dot(w, wt[...], preferred_element_type=jnp.float32) + bs[...]
                i = jax.nn.sigmoid(g[:, 0:H])
                f = jax.nn.sigmoid(g[:, H:2 * H])
                gg = jnp.tanh(g[:, 2 * H:3 * H])
                o = jax.nn.sigmoid(g[:, 3 * H:4 * H])
                c = f * c + i * gg
                w = o * jnp.tanh(c)
                o_ref[li, t] = w

    return pl.pallas_call(
        body,
        out_shape=jax.ShapeDtypeStruct((2, TP, H, H), jnp.float32),
    )(w1_0, w1_1, r1_WihT, r2_WihT, b1sum, b2sum)


def _tc_stage_a(x3, Wg, dinv3):
    def body(x_ref, w_ref, d_ref, o_ref):
        o_ref[0] = jnp.dot(x_ref[0], w_ref[...],
                           preferred_element_type=jnp.float32) * d_ref[0]

    return pl.pallas_call(
        body,
        grid=(TP, NB),
        in_specs=[
            pl.BlockSpec((1, BLK, H), lambda t, b: (t, b, 0)),
            pl.BlockSpec((H, H), lambda t, b: (0, 0)),
            pl.BlockSpec((1, BLK, 1), lambda t, b: (t, b, 0)),
        ],
        out_specs=pl.BlockSpec((1, BLK, H), lambda t, b: (t, b, 0)),
        out_shape=jax.ShapeDtypeStruct((TP, N, H), jnp.float32),
    )(x3, Wg, dinv3)


def _tc_stage_b(Sp, hs, dinv3, bias):
    def body(s_ref, h_ref, d_ref, b_ref, o_ref):
        o_ref[0] = (s_ref[0, 0] + s_ref[1, 0] + h_ref[0]) * d_ref[0] + b_ref[...]

    return pl.pallas_call(
        body,
        grid=(TP, NB),
        in_specs=[
            pl.BlockSpec((2, 1, BLK, H), lambda t, b: (0, t, b, 0)),
            pl.BlockSpec((1, BLK, H), lambda t, b: (t, b, 0)),
            pl.BlockSpec((1, BLK, 1), lambda t, b: (t, b, 0)),
            pl.BlockSpec((1, H), lambda t, b: (0, 0)),
        ],
        out_specs=pl.BlockSpec((1, BLK, H), lambda t, b: (t, b, 0)),
        out_shape=jax.ShapeDtypeStruct((TP, N, H), jnp.float32),
    )(Sp, hs, dinv3, bias)


def _tc_stage_c1(Ap, z, w1t):
    def body(a_ref, z_ref, w_ref, o_ref, ps_ref, pss_ref):
        zv = z_ref[0]
        out1 = 0.9 * (a_ref[0, 0] + a_ref[1, 0]) + 0.1 * zv
        z1 = (1.0 - BETA1) * out1 + BETA1 * jnp.dot(
            out1, w_ref[0], preferred_element_type=jnp.float32)
        o_ref[0] = z1
        ps_ref[0, 0, 0] = jnp.sum(z1, axis=0)
        pss_ref[0, 0, 0] = jnp.sum(z1 * z1, axis=0)

    return pl.pallas_call(
        body,
        grid=(TP, NB),
        in_specs=[
            pl.BlockSpec((2, 1, BLK, H), lambda t, b: (0, t, b, 0)),
            pl.BlockSpec((1, BLK, H), lambda t, b: (t, b, 0)),
            pl.BlockSpec((1, H, H), lambda t, b: (t, 0, 0)),
        ],
        out_specs=[
            pl.BlockSpec((1, BLK, H), lambda t, b: (t, b, 0)),
            pl.BlockSpec((1, 1, 1, H), lambda t, b: (t, b, 0, 0)),
            pl.BlockSpec((1, 1, 1, H), lambda t, b: (t, b, 0, 0)),
        ],
        out_shape=[
            jax.ShapeDtypeStruct((TP, N, H), jnp.float32),
            jax.ShapeDtypeStruct((TP, NB, 1, H), jnp.float32),
            jax.ShapeDtypeStruct((TP, NB, 1, H), jnp.float32),
        ],
    )(Ap, z, w1t)


def _tc_stage_c2(z1, mu, var, gamma, beta):
    def body(z_ref, m_ref, v_ref, g_ref, b_ref, o_ref):
        scale = lax.rsqrt(v_ref[0] + 1e-5) * g_ref[...]
        o_ref[0] = jax.nn.relu((z_ref[0] - m_ref[0]) * scale + b_ref[...])

    return pl.pallas_call(
        body,
        grid=(TP, NB),
        in_specs=[
            pl.BlockSpec((1, BLK, H), lambda t, b: (t, b, 0)),
            pl.BlockSpec((1, 1, H), lambda t, b: (t, 0, 0)),
            pl.BlockSpec((1, 1, H), lambda t, b: (t, 0, 0)),
            pl.BlockSpec((1, H), lambda t, b: (0, 0)),
            pl.BlockSpec((1, H), lambda t, b: (0, 0)),
        ],
        out_specs=pl.BlockSpec((1, BLK, H), lambda t, b: (t, b, 0)),
        out_shape=jax.ShapeDtypeStruct((TP, N, H), jnp.float32),
    )(z1, mu, var, gamma, beta)


def _tc_stage_d(Ap, x0, w1t):
    def body(a_ref, x_ref, w_ref, o_ref):
        out2 = 0.9 * (a_ref[0, 0] + a_ref[1, 0]) + 0.1 * x_ref[0]
        o_ref[0] = (1.0 - BETA2) * out2 + BETA2 * jnp.dot(
            out2, w_ref[0], preferred_element_type=jnp.float32)

    return pl.pallas_call(
        body,
        grid=(TP, NB),
        in_specs=[
            pl.BlockSpec((2, 1, BLK, H), lambda t, b: (0, t, b, 0)),
            pl.BlockSpec((1, BLK, H), lambda t, b: (t, b, 0)),
            pl.BlockSpec((1, H, H), lambda t, b: (t, 0, 0)),
        ],
        out_specs=pl.BlockSpec((1, BLK, H), lambda t, b: (t, b, 0)),
        out_shape=jax.ShapeDtypeStruct((TP, N, H), jnp.float32),
    )(Ap, x0, w1t)


def _tc_lstm(z2, WihT, WhhT, bsum):
    def body(za_ref, zb_ref, wi_ref, wh_ref, b_ref, o_ref):
        wi = wi_ref[...]
        wh = wh_ref[...]
        bs = b_ref[...]
        g = jnp.dot(za_ref[0], wi, preferred_element_type=jnp.float32) + bs
        i = jax.nn.sigmoid(g[:, 0:H])
        gg = jnp.tanh(g[:, 2 * H:3 * H])
        o = jax.nn.sigmoid(g[:, 3 * H:4 * H])
        c = i * gg
        h = o * jnp.tanh(c)
        g = (jnp.dot(zb_ref[0], wi, preferred_element_type=jnp.float32)
             + jnp.dot(h, wh, preferred_element_type=jnp.float32) + bs)
        i = jax.nn.sigmoid(g[:, 0:H])
        f = jax.nn.sigmoid(g[:, H:2 * H])
        gg = jnp.tanh(g[:, 2 * H:3 * H])
        o = jax.nn.sigmoid(g[:, 3 * H:4 * H])
        c = f * c + i * gg
        h = o * jnp.tanh(c)
        g = jnp.dot(h, wi, preferred_element_type=jnp.float32) + bs
        i = jax.nn.sigmoid(g[:, 0:H])
        f = jax.nn.sigmoid(g[:, H:2 * H])
        gg = jnp.tanh(g[:, 2 * H:3 * H])
        o = jax.nn.sigmoid(g[:, 3 * H:4 * H])
        c = f * c + i * gg
        o_ref[...] = o * jnp.tanh(c)

    return pl.pallas_call(
        body,
        grid=(NB,),
        in_specs=[
            pl.BlockSpec((1, BLK, H), lambda b: (0, b, 0)),
            pl.BlockSpec((1, BLK, H), lambda b: (1, b, 0)),
            pl.BlockSpec((H, 4 * H), lambda b: (0, 0)),
            pl.BlockSpec((H, 4 * H), lambda b: (0, 0)),
            pl.BlockSpec((1, 4 * H), lambda b: (0, 0)),
        ],
        out_specs=pl.BlockSpec((BLK, H), lambda b: (b, 0)),
        out_shape=jax.ShapeDtypeStruct((N, H), jnp.float32),
    )(z2, z2, WihT, WhhT, bsum)


def kernel(x_seq, edge_index_seq, W_gcn, b_gcn, w1_0, w1_1,
           r0_Wih, r0_Whh, r0_bih, r0_bhh,
           r1_Wih, r1_Whh, r1_bih, r1_bhh,
           r2_Wih, r2_Whh, r2_bih, r2_bhh,
           f_Wih, f_Whh, f_bih, f_bhh, bn_gamma, bn_beta):
    f32 = jnp.float32
    xs = x_seq[:TP]
    src = edge_index_seq[:TP, 0]
    dst = edge_index_seq[:TP, 1]

    pad = EP - E
    toff = (jnp.arange(TP, dtype=jnp.int32) * N)[:, None]
    src_g = (jnp.pad(src, ((0, 0), (0, pad))) + toff).reshape(TP, NCH, C)
    pad_ids = N + (jnp.arange(pad, dtype=jnp.int32) % (NP - N))
    dst_p = jnp.concatenate(
        [dst, jnp.broadcast_to(pad_ids, (TP, pad))], axis=1)
    dst_g = dst_p.reshape(TP, NCH, C)
    doff = (jnp.arange(TP, dtype=jnp.int32) * NP)[:, None]
    dst_deg = (dst_p + doff).reshape(TP, NCH, C)

    ones_c = jnp.ones((C,), f32)
    zeros_deg = jnp.zeros((DEGT,), f32)
    zeros_rows = jnp.zeros((ZROWS, H), f32)

    degp = _sc_degrees(dst_deg, ones_c, zeros_deg)
    deg = (degp[0, 0] + degp[1, 0])[:TP * NP].reshape(TP, NP)[:, :N] + 1.0
    dinv3 = lax.rsqrt(deg)[:, :, None]

    w1s = _tc_weights(w1_0, w1_1, jnp.transpose(r1_Wih),
                      jnp.transpose(r2_Wih),
                      (r1_bih + r1_bhh)[None, :], (r2_bih + r2_bhh)[None, :])

    hs = _tc_stage_a(xs, W_gcn, dinv3)
    Sp = _sc_segsum(hs.reshape(TP * N, H), src_g, dst_g, zeros_rows)
    z = _tc_stage_b(Sp, hs, dinv3, b_gcn[None, :])

    A1p = _sc_segsum(z.reshape(TP * N, H), src_g, dst_g, zeros_rows)
    z1, ps, pss = _tc_stage_c1(A1p, z, w1s[0])
    mu = ps.sum(axis=1) / N
    var = pss.sum(axis=1) / N - mu * mu
    z1n = _tc_stage_c2(z1, mu, var, bn_gamma[None, :], bn_beta[None, :])

    A2p = _sc_segsum(z1n.reshape(TP * N, H), src_g, dst_g, zeros_rows)
    z2 = _tc_stage_d(A2p, z, w1s[1])

    h2 = _tc_lstm(z2, jnp.transpose(f_Wih), jnp.transpose(f_Whh),
                  (f_bih + f_bhh)[None, :])

    return jnp.concatenate([z2[:2], h2[None]], axis=0)
